# baseline (speedup 1.0000x reference)
import functools

import jax
import jax.numpy as jnp
from jax import lax
from jax.experimental import pallas as pl
from jax.experimental.pallas import tpu as pltpu
from jax.experimental.pallas import tpu_sc as plsc

H = 128
L = 16
NC = 2
NS = 16
NW = NC * NS
BATCH = 4
SEQ = 2048
PSLICE = 128
CHUNK = 32
NCHUNK = 8
NSLICE = SEQ // PSLICE

_mesh = plsc.VectorSubcoreMesh(core_axis_name="c", subcore_axis_name="s")


@functools.partial(
    pl.kernel,
    out_type=jax.ShapeDtypeStruct((BATCH, SEQ, H), jnp.float32),
    mesh=_mesh,
    scratch_types=[
        pltpu.VMEM((PSLICE,), jnp.int32),
        pltpu.VMEM((PSLICE,), jnp.int32),
        pltpu.VMEM((PSLICE, H), jnp.float32),
        [pltpu.VMEM((CHUNK, H), jnp.float32) for _ in range(NCHUNK)],
        [pltpu.SemaphoreType.DMA for _ in range(NCHUNK)],
        pltpu.SemaphoreType.DMA,
        pltpu.SemaphoreType.DMA,
    ],
)
def _embed_lookup(x_hbm, tok_hbm, pos_hbm, out_hbm,
                  idxa_v, idxb_v, pos_v, tok_bufs, g_sems, idx_sem, out_sem):
    wid = lax.axis_index("s") * NC + lax.axis_index("c")
    b0 = (wid // NSLICE) * 2
    s1 = (wid % NSLICE) * PSLICE

    ia = pltpu.async_copy(x_hbm.at[b0, pl.ds(s1, PSLICE)], idxa_v, idx_sem)
    ib = pltpu.async_copy(x_hbm.at[b0 + 1, pl.ds(s1, PSLICE)], idxb_v, idx_sem)
    ia.wait()
    g = [None] * NCHUNK
    for k in range(NCHUNK // 2):
        g[k] = pltpu.async_copy(
            tok_hbm.at[idxa_v.at[pl.ds(k * CHUNK, CHUNK)]], tok_bufs[k], g_sems[k])
    ib.wait()
    for k in range(NCHUNK // 2, NCHUNK):
        g[k] = pltpu.async_copy(
            tok_hbm.at[idxb_v.at[pl.ds((k - NCHUNK // 2) * CHUNK, CHUNK)]],
            tok_bufs[k], g_sems[k])
    pltpu.sync_copy(pos_hbm.at[pl.ds(s1, PSLICE), :], pos_v)

    def add_rows(tok_ref, pos_off):
        @plsc.parallel_loop(0, CHUNK, unroll=2)
        def body(j):
            for c in range(H // L):
                sl = pl.ds(c * L, L)
                plsc.addupdate(tok_ref.at[j, sl], pos_v[pos_off + j, sl])

    outs = []
    for k in range(NCHUNK):
        g[k].wait()
        add_rows(tok_bufs[k], (k % 4) * CHUNK)
        dst = out_hbm.at[b0 + k // 4, pl.ds(s1 + (k % 4) * CHUNK, CHUNK), :]
        outs.append(pltpu.async_copy(tok_bufs[k], dst, out_sem))
    for o in outs:
        o.wait()


def kernel(x, token_table, position_table):
    return _embed_lookup(x.astype(jnp.int32), token_table, position_table)


# speedup vs baseline: 1.0302x; 1.0302x over previous
import functools

import jax
import jax.numpy as jnp
from jax import lax
from jax.experimental import pallas as pl
from jax.experimental.pallas import tpu as pltpu
from jax.experimental.pallas import tpu_sc as plsc

H = 128
L = 16
NC = 2
NS = 16
NW = NC * NS
BATCH = 4
SEQ = 2048
PSLICE = SEQ // NW

_mesh = plsc.VectorSubcoreMesh(core_axis_name="c", subcore_axis_name="s")


@functools.partial(
    pl.kernel,
    out_type=jax.ShapeDtypeStruct((BATCH, SEQ, H), jnp.float32),
    mesh=_mesh,
    scratch_types=[
        [pltpu.VMEM((PSLICE,), jnp.int32) for _ in range(BATCH)],
        pltpu.VMEM((PSLICE, H), jnp.float32),
        [pltpu.VMEM((PSLICE, H), jnp.float32) for _ in range(BATCH)],
        [pltpu.SemaphoreType.DMA for _ in range(BATCH)],
        pltpu.SemaphoreType.DMA,
        pltpu.SemaphoreType.DMA,
    ],
)
def _embed_lookup(x_hbm, tok_hbm, pos_hbm, out_hbm,
                  idx_bufs, pos_v, tok_bufs, g_sems, idx_sem, out_sem):
    wid = lax.axis_index("s") * NC + lax.axis_index("c")
    s1 = wid * PSLICE

    idx_copies = [
        pltpu.async_copy(x_hbm.at[b, pl.ds(s1, PSLICE)], idx_bufs[b], idx_sem)
        for b in range(BATCH)
    ]
    g = []
    for b in range(BATCH):
        idx_copies[b].wait()
        g.append(pltpu.async_copy(tok_hbm.at[idx_bufs[b]], tok_bufs[b], g_sems[b]))
    pltpu.sync_copy(pos_hbm.at[pl.ds(s1, PSLICE), :], pos_v)

    def add_rows(tok_ref):
        @plsc.parallel_loop(0, PSLICE, unroll=2)
        def body(j):
            for c in range(H // L):
                sl = pl.ds(c * L, L)
                plsc.addupdate(tok_ref.at[j, sl], pos_v[j, sl])

    outs = []
    for b in range(BATCH):
        g[b].wait()
        add_rows(tok_bufs[b])
        dst = out_hbm.at[b, pl.ds(s1, PSLICE), :]
        outs.append(pltpu.async_copy(tok_bufs[b], dst, out_sem))
    for o in outs:
        o.wait()


def kernel(x, token_table, position_table):
    return _embed_lookup(x.astype(jnp.int32), token_table, position_table)
